# baseline (speedup 1.0000x reference)
import jax
import jax.numpy as jnp
from jax.experimental import pallas as pl
from jax.experimental.pallas import tpu as pltpu

B, N, D, E = 16, 512, 32, 4
T = 256
NT = N // T


def _ggnn_kernel(xT_ref, edges_hbm, WeT_ref, Wzr_ref, Whu_ref, Wo_ref,
                 bzT_ref, brT_ref, bhT_ref, boT_ref,
                 out_ref, ebuf, stage, txbuf, sems):
    b = pl.program_id(0)
    p = jax.lax.rem(b, 2)
    xT = xT_ref[0]

    @pl.when(b == 0)
    def _prologue():
        for e in range(E):
            pltpu.make_async_copy(
                edges_hbm.at[0, e], ebuf.at[0, e], sems.at[0, e]).start()

    @pl.when(b < B - 1)
    def _prefetch():
        for e in range(E):
            pltpu.make_async_copy(
                edges_hbm.at[b + 1, e], ebuf.at[1 - p, e],
                sems.at[1 - p, e]).start()

    stage[0:D, :] = xT
    for kt in range(NT):
        pltpu.matmul_push_rhs(stage[:, kt * T:(kt + 1) * T], 0, kt,
                              transpose=False)
        for e in range(E):
            pltpu.matmul_acc_lhs(0, WeT_ref[e], kt,
                                 load_staged_rhs=0 if e == 0 else None)
            txbuf[e, :, kt * T:(kt + 1) * T] = pltpu.matmul_pop(
                0, (D, T), jnp.float32, kt)

    for e in range(E):
        pltpu.make_async_copy(
            edges_hbm.at[b, e], ebuf.at[p, e], sems.at[p, e]).wait()
    parts = []
    for nt in range(NT):
        reg = 0
        for e in range(E):
            for kt in range(NT):
                rhs = ebuf[p, e, nt * T:(nt + 1) * T,
                           kt * T:(kt + 1) * T].astype(jnp.bfloat16)
                pltpu.matmul_push_rhs(rhs, reg, nt, transpose=True)
                pltpu.matmul_acc_lhs(0, txbuf[e, :, kt * T:(kt + 1) * T],
                                     nt, load_staged_rhs=reg)
                reg = 1 - reg
        parts.append(pltpu.matmul_pop(0, (D, T), jnp.float32, nt))
    mT = jnp.concatenate(parts, axis=1)

    stage[0:D, :] = mT
    stage[D:2 * D, :] = xT
    parts = []
    for nt in range(NT):
        pltpu.matmul_push_rhs(stage[:, nt * T:(nt + 1) * T], 0, nt,
                              transpose=False)
        pltpu.matmul_acc_lhs(0, Wzr_ref[...], nt, load_staged_rhs=0)
        parts.append(pltpu.matmul_pop(0, (2 * D, T), jnp.float32, nt))
    zr = jnp.concatenate(parts, axis=1)
    z = jax.nn.sigmoid(zr[0:D] + bzT_ref[...])
    r = jax.nn.sigmoid(zr[D:2 * D] + brT_ref[...])

    stage[D:2 * D, :] = r * xT
    parts = []
    for nt in range(NT):
        pltpu.matmul_push_rhs(stage[:, nt * T:(nt + 1) * T], 0, nt,
                              transpose=False)
        pltpu.matmul_acc_lhs(0, Whu_ref[...], nt, load_staged_rhs=0)
        parts.append(pltpu.matmul_pop(0, (D, T), jnp.float32, nt))
    h_til = jnp.tanh(jnp.concatenate(parts, axis=1) + bhT_ref[...])
    hT = (1.0 - z) * xT + z * h_til

    stage[0:D, :] = hT
    parts = []
    for nt in range(NT):
        pltpu.matmul_push_rhs(stage[:, nt * T:(nt + 1) * T], 0, nt,
                              transpose=False)
        pltpu.matmul_acc_lhs(0, Wo_ref[...], nt, load_staged_rhs=0)
        parts.append(pltpu.matmul_pop(0, (8, T), jnp.float32, nt))
    logits = jnp.concatenate(parts, axis=1)[0:5] + boT_ref[...]
    lmax = jnp.max(logits, axis=0, keepdims=True)
    shifted = logits - lmax
    lse = jnp.log(jnp.sum(jnp.exp(shifted), axis=0, keepdims=True))
    out_ref[0] = shifted - lse


@jax.jit
def kernel(x_padded, x_lengths, edges, fingers, W_edge, Wz, Uz, bz,
           Wr, Ur, br, Wh, Uh, bh, W_out, b_out):
    del x_lengths, fingers
    f32 = jnp.float32

    WeT_pad = jnp.zeros((E, D, T), f32).at[:, :, 0:D].set(
        W_edge.transpose(0, 2, 1))
    Wzr_pad = (jnp.zeros((2 * D, T), f32)
               .at[0:D, 0:D].set(Wz.T).at[0:D, D:2 * D].set(Uz.T)
               .at[D:2 * D, 0:D].set(Wr.T).at[D:2 * D, D:2 * D].set(Ur.T))
    Whu_pad = (jnp.zeros((D, T), f32)
               .at[:, 0:D].set(Wh.T).at[:, D:2 * D].set(Uh.T))
    Wo_pad = jnp.zeros((8, T), f32).at[0:5, 0:D].set(W_out.T)

    full = lambda b: (0, 0)
    outT = pl.pallas_call(
        _ggnn_kernel,
        grid=(B,),
        in_specs=[
            pl.BlockSpec((1, D, N), lambda b: (b, 0, 0)),
            pl.BlockSpec(memory_space=pltpu.MemorySpace.HBM),
            pl.BlockSpec((E, D, T), lambda b: (0, 0, 0)),
            pl.BlockSpec((2 * D, T), full),
            pl.BlockSpec((D, T), full),
            pl.BlockSpec((8, T), full),
            pl.BlockSpec((D, 1), full),
            pl.BlockSpec((D, 1), full),
            pl.BlockSpec((D, 1), full),
            pl.BlockSpec((5, 1), full),
        ],
        out_specs=pl.BlockSpec((1, 5, N), lambda b: (b, 0, 0)),
        out_shape=jax.ShapeDtypeStruct((B, 5, N), jnp.float32),
        scratch_shapes=[
            pltpu.VMEM((2, E, N, N), jnp.float32),
            pltpu.VMEM((T, N), jnp.float32),
            pltpu.VMEM((E, D, N), jnp.float32),
            pltpu.SemaphoreType.DMA((2, E)),
        ],
        compiler_params=pltpu.CompilerParams(
            dimension_semantics=("arbitrary",)),
    )(x_padded.transpose(0, 2, 1), edges, WeT_pad, Wzr_pad, Whu_pad, Wo_pad,
      bz.reshape(D, 1), br.reshape(D, 1), bh.reshape(D, 1),
      b_out.reshape(5, 1))
    return outT.transpose(0, 2, 1)


# speedup vs baseline: 1.2903x; 1.2903x over previous
import jax
import jax.numpy as jnp
from jax.experimental import pallas as pl
from jax.experimental.pallas import tpu as pltpu

B, N, D, E = 16, 512, 32, 4
BB = 2


def _ggnn_kernel(xT_ref, edges_ref, WeT_ref, WzT_ref, UzT_ref, bzT_ref,
                 WrT_ref, UrT_ref, brT_ref, WhT_ref, UhT_ref, bhT_ref,
                 WoT_ref, boT_ref, out_ref):
    for bb in range(BB):
        xT = xT_ref[bb]

        mT = jnp.zeros((D, N), dtype=jnp.float32)
        for e in range(E):
            txT = jnp.dot(WeT_ref[e], xT, preferred_element_type=jnp.float32)
            mT = mT + jax.lax.dot_general(
                txT, edges_ref[bb, e],
                dimension_numbers=(((1,), (1,)), ((), ())),
                precision=jax.lax.Precision.DEFAULT,
                preferred_element_type=jnp.float32)

        z = jax.nn.sigmoid(jnp.dot(WzT_ref[...], mT)
                           + jnp.dot(UzT_ref[...], xT) + bzT_ref[...])
        r = jax.nn.sigmoid(jnp.dot(WrT_ref[...], mT)
                           + jnp.dot(UrT_ref[...], xT) + brT_ref[...])
        h_til = jnp.tanh(jnp.dot(WhT_ref[...], mT)
                         + jnp.dot(UhT_ref[...], r * xT) + bhT_ref[...])
        hT = (1.0 - z) * xT + z * h_til

        logits = jnp.dot(WoT_ref[...], hT) + boT_ref[...]
        lmax = jnp.max(logits, axis=0, keepdims=True)
        shifted = logits - lmax
        lse = jnp.log(jnp.sum(jnp.exp(shifted), axis=0, keepdims=True))
        out_ref[bb] = shifted - lse


@jax.jit
def kernel(x_padded, x_lengths, edges, fingers, W_edge, Wz, Uz, bz,
           Wr, Ur, br, Wh, Uh, bh, W_out, b_out):
    del x_lengths, fingers
    full = lambda g: (0, 0)
    outT = pl.pallas_call(
        _ggnn_kernel,
        grid=(B // BB,),
        in_specs=[
            pl.BlockSpec((BB, D, N), lambda g: (g, 0, 0)),
            pl.BlockSpec((BB, E, N, N), lambda g: (g, 0, 0, 0)),
            pl.BlockSpec((E, D, D), lambda g: (0, 0, 0)),
            pl.BlockSpec((D, D), full),
            pl.BlockSpec((D, D), full),
            pl.BlockSpec((D, 1), full),
            pl.BlockSpec((D, D), full),
            pl.BlockSpec((D, D), full),
            pl.BlockSpec((D, 1), full),
            pl.BlockSpec((D, D), full),
            pl.BlockSpec((D, D), full),
            pl.BlockSpec((D, 1), full),
            pl.BlockSpec((5, D), full),
            pl.BlockSpec((5, 1), full),
        ],
        out_specs=pl.BlockSpec((BB, 5, N), lambda g: (g, 0, 0)),
        out_shape=jax.ShapeDtypeStruct((B, 5, N), jnp.float32),
        compiler_params=pltpu.CompilerParams(
            dimension_semantics=("arbitrary",)),
    )(x_padded.transpose(0, 2, 1), edges,
      W_edge.transpose(0, 2, 1),
      Wz.T, Uz.T, bz.reshape(D, 1),
      Wr.T, Ur.T, br.reshape(D, 1),
      Wh.T, Uh.T, bh.reshape(D, 1),
      W_out.T, b_out.reshape(5, 1))
    return outT.transpose(0, 2, 1)


# speedup vs baseline: 1.2921x; 1.0014x over previous
import jax
import jax.numpy as jnp
from jax.experimental import pallas as pl
from jax.experimental.pallas import tpu as pltpu

B, N, D, E = 16, 512, 32, 4
BB = 4


def _ggnn_kernel(xT_ref, edges_ref, WeT_ref, WzT_ref, UzT_ref, bzT_ref,
                 WrT_ref, UrT_ref, brT_ref, WhT_ref, UhT_ref, bhT_ref,
                 WoT_ref, boT_ref, out_ref):
    for bb in range(BB):
        xT = xT_ref[bb]

        mT = jnp.zeros((D, N), dtype=jnp.float32)
        for e in range(E):
            txT = jnp.dot(WeT_ref[e], xT, preferred_element_type=jnp.float32)
            mT = mT + jax.lax.dot_general(
                txT, edges_ref[bb, e],
                dimension_numbers=(((1,), (1,)), ((), ())),
                precision=jax.lax.Precision.DEFAULT,
                preferred_element_type=jnp.float32)

        z = jax.nn.sigmoid(jnp.dot(WzT_ref[...], mT)
                           + jnp.dot(UzT_ref[...], xT) + bzT_ref[...])
        r = jax.nn.sigmoid(jnp.dot(WrT_ref[...], mT)
                           + jnp.dot(UrT_ref[...], xT) + brT_ref[...])
        h_til = jnp.tanh(jnp.dot(WhT_ref[...], mT)
                         + jnp.dot(UhT_ref[...], r * xT) + bhT_ref[...])
        hT = (1.0 - z) * xT + z * h_til

        logits = jnp.dot(WoT_ref[...], hT) + boT_ref[...]
        lmax = jnp.max(logits, axis=0, keepdims=True)
        shifted = logits - lmax
        lse = jnp.log(jnp.sum(jnp.exp(shifted), axis=0, keepdims=True))
        out_ref[bb] = shifted - lse


@jax.jit
def kernel(x_padded, x_lengths, edges, fingers, W_edge, Wz, Uz, bz,
           Wr, Ur, br, Wh, Uh, bh, W_out, b_out):
    del x_lengths, fingers
    full = lambda g: (0, 0)
    outT = pl.pallas_call(
        _ggnn_kernel,
        grid=(B // BB,),
        in_specs=[
            pl.BlockSpec((BB, D, N), lambda g: (g, 0, 0)),
            pl.BlockSpec((BB, E, N, N), lambda g: (g, 0, 0, 0)),
            pl.BlockSpec((E, D, D), lambda g: (0, 0, 0)),
            pl.BlockSpec((D, D), full),
            pl.BlockSpec((D, D), full),
            pl.BlockSpec((D, 1), full),
            pl.BlockSpec((D, D), full),
            pl.BlockSpec((D, D), full),
            pl.BlockSpec((D, 1), full),
            pl.BlockSpec((D, D), full),
            pl.BlockSpec((D, D), full),
            pl.BlockSpec((D, 1), full),
            pl.BlockSpec((5, D), full),
            pl.BlockSpec((5, 1), full),
        ],
        out_specs=pl.BlockSpec((BB, 5, N), lambda g: (g, 0, 0)),
        out_shape=jax.ShapeDtypeStruct((B, 5, N), jnp.float32),
        compiler_params=pltpu.CompilerParams(
            dimension_semantics=("arbitrary",)),
    )(x_padded.transpose(0, 2, 1), edges,
      W_edge.transpose(0, 2, 1),
      Wz.T, Uz.T, bz.reshape(D, 1),
      Wr.T, Ur.T, br.reshape(D, 1),
      Wh.T, Uh.T, bh.reshape(D, 1),
      W_out.T, b_out.reshape(5, 1))
    return outT.transpose(0, 2, 1)
